# Initial kernel scaffold; baseline (speedup 1.0000x reference)
import jax
import jax.numpy as jnp
from jax.experimental import pallas as pl


def kernel(x, edge_index, edge_attr, u, batch, W1, b1, g1, be1, W2, b2, g2, be2, W3, b3):
    raise NotImplementedError("write your pallas kernel here")


# speedup vs baseline: 3.4681x; 3.4681x over previous
import functools

import jax
import jax.numpy as jnp
from jax import lax
from jax.experimental import pallas as pl
from jax.experimental.pallas import tpu as pltpu
from jax.experimental.pallas import tpu_sc as plsc

_NUM_CORES = 2
_NUM_SUBCORES = 16
_NW = _NUM_CORES * _NUM_SUBCORES
_CHUNK = 128


def _sc_scatter_partials(dst_idx, edge_attr, n_pad):
    e, d_edge = edge_attr.shape
    n_chunks = e // _CHUNK
    rows_per_sub = n_pad // _NUM_SUBCORES
    chunks_per, chunks_rem = divmod(n_chunks, _NW)

    mesh = plsc.VectorSubcoreMesh(
        core_axis_name="c", subcore_axis_name="s",
        num_cores=_NUM_CORES, num_subcores=_NUM_SUBCORES)

    @functools.partial(
        pl.kernel,
        out_type=jax.ShapeDtypeStruct((_NUM_CORES, n_pad, d_edge),
                                      jnp.float32),
        mesh=mesh,
        scratch_types=[
            pltpu.VMEM_SHARED((n_pad, d_edge), jnp.float32),
            pltpu.VMEM((_CHUNK,), jnp.int32),
            pltpu.VMEM((_CHUNK, d_edge), jnp.float32),
            pltpu.VMEM((rows_per_sub, d_edge), jnp.float32),
        ],
    )
    def scatter_kernel(dst_hbm, ea_hbm, out_hbm, agg_sh, idx_v, rows_v, zbuf):
        cid = lax.axis_index("c")
        sid = lax.axis_index("s")
        wid = cid * _NUM_SUBCORES + sid

        zero_row = jnp.zeros((d_edge,), jnp.float32)

        def zero_body(i, carry):
            zbuf[i] = zero_row
            return carry

        lax.fori_loop(0, rows_per_sub, zero_body, 0)
        row0 = sid * rows_per_sub
        pltpu.sync_copy(zbuf, agg_sh.at[pl.ds(row0, rows_per_sub)])
        plsc.subcore_barrier()

        start = wid * chunks_per + jnp.minimum(wid, chunks_rem)
        count = chunks_per + jnp.where(wid < chunks_rem, 1, 0)

        def chunk_body(j, carry):
            pltpu.sync_copy(dst_hbm.at[pl.ds(j * _CHUNK, _CHUNK)], idx_v)
            pltpu.sync_copy(ea_hbm.at[pl.ds(j * _CHUNK, _CHUNK)], rows_v)
            pltpu.sync_copy(rows_v, agg_sh.at[idx_v], add=True)
            return carry

        lax.fori_loop(start, start + count, chunk_body, 0)
        plsc.subcore_barrier()

        pltpu.sync_copy(agg_sh.at[pl.ds(row0, rows_per_sub)],
                        out_hbm.at[cid, pl.ds(row0, rows_per_sub)])

    return scatter_kernel(dst_idx, edge_attr)


def _mlp_body(x_ref, a0_ref, a1_ref, u_ref, w1x_ref, w1a_ref, w1u_ref,
              b1_ref, g1_ref, be1_ref, w2_ref, b2_ref, g2_ref, be2_ref,
              w3_ref, b3_ref, out_ref):
    f32 = jnp.float32
    agg = a0_ref[...] + a1_ref[...]
    h = (jnp.dot(x_ref[...], w1x_ref[...], preferred_element_type=f32)
         + jnp.dot(agg, w1a_ref[...], preferred_element_type=f32)
         + jnp.dot(u_ref[...], w1u_ref[...], preferred_element_type=f32)
         + b1_ref[...])
    h = jnp.maximum(h, 0.0)
    mu = jnp.mean(h, axis=-1, keepdims=True)
    var = jnp.mean((h - mu) ** 2, axis=-1, keepdims=True)
    h = (h - mu) / jnp.sqrt(var + 1e-5) * g1_ref[...] + be1_ref[...]
    h = jnp.dot(h, w2_ref[...], preferred_element_type=f32) + b2_ref[...]
    h = jnp.maximum(h, 0.0)
    mu = jnp.mean(h, axis=-1, keepdims=True)
    var = jnp.mean((h - mu) ** 2, axis=-1, keepdims=True)
    h = (h - mu) / jnp.sqrt(var + 1e-5) * g2_ref[...] + be2_ref[...]
    out_ref[...] = (jnp.dot(h, w3_ref[...], preferred_element_type=f32)
                    + b3_ref[...])


def kernel(x, edge_index, edge_attr, u, batch, W1, b1, g1, be1,
           W2, b2, g2, be2, W3, b3):
    n, d_feat = x.shape
    d_edge = edge_attr.shape[1]
    d_u = u.shape[1]
    hidden = W1.shape[1]
    out_size = W3.shape[1]

    dst_idx = edge_index[1].astype(jnp.int32)
    n_pad = -(-n // 128) * 128
    partials = _sc_scatter_partials(dst_idx, edge_attr, n_pad)
    a0, a1 = partials[0, :n], partials[1, :n]

    w1x = W1[:d_feat]
    w1a = W1[d_feat:d_feat + d_edge]
    w1u = W1[d_feat + d_edge:]

    br = 1000 if n % 1000 == 0 else n
    grid = (n // br,)
    row_spec = lambda width: pl.BlockSpec((br, width), lambda i: (i, 0))
    full = lambda a: pl.BlockSpec(a.shape, lambda i: (0,) * a.ndim)

    b1r, g1r, be1r = b1.reshape(1, -1), g1.reshape(1, -1), be1.reshape(1, -1)
    b2r, g2r, be2r = b2.reshape(1, -1), g2.reshape(1, -1), be2.reshape(1, -1)
    b3r = b3.reshape(1, -1)

    return pl.pallas_call(
        _mlp_body,
        grid=grid,
        in_specs=[
            row_spec(d_feat), row_spec(d_edge), row_spec(d_edge),
            full(u), full(w1x), full(w1a), full(w1u),
            full(b1r), full(g1r), full(be1r),
            full(W2), full(b2r), full(g2r), full(be2r),
            full(W3), full(b3r),
        ],
        out_specs=pl.BlockSpec((br, out_size), lambda i: (i, 0)),
        out_shape=jax.ShapeDtypeStruct((n, out_size), jnp.float32),
    )(x, a0, a1, u, w1x, w1a, w1u, b1r, g1r, be1r,
      W2, b2r, g2r, be2r, W3, b3r)


# speedup vs baseline: 3.7219x; 1.0732x over previous
import functools

import jax
import jax.numpy as jnp
from jax import lax
from jax.experimental import pallas as pl
from jax.experimental.pallas import tpu as pltpu
from jax.experimental.pallas import tpu_sc as plsc

_NUM_CORES = 2
_NUM_SUBCORES = 16
_NW = _NUM_CORES * _NUM_SUBCORES
_CHUNK = 128


def _sc_scatter_partials(dst_idx, edge_attr, n_pad):
    e, d_edge = edge_attr.shape
    n_chunks = e // _CHUNK
    rows_per_sub = n_pad // _NUM_SUBCORES
    chunks_per, chunks_rem = divmod(n_chunks, _NW)

    mesh = plsc.VectorSubcoreMesh(
        core_axis_name="c", subcore_axis_name="s",
        num_cores=_NUM_CORES, num_subcores=_NUM_SUBCORES)

    pairs = chunks_per // 2
    odd = chunks_per % 2

    @functools.partial(
        pl.kernel,
        out_type=jax.ShapeDtypeStruct((_NUM_CORES, n_pad, d_edge),
                                      jnp.float32),
        mesh=mesh,
        compiler_params=pltpu.CompilerParams(use_tc_tiling_on_sc=False),
        scratch_types=[
            pltpu.VMEM_SHARED((n_pad, d_edge), jnp.float32),
            pltpu.VMEM((_CHUNK,), jnp.int32),
            pltpu.VMEM((_CHUNK, d_edge), jnp.float32),
            pltpu.VMEM((rows_per_sub, d_edge), jnp.float32),
        ],
    )
    def scatter_kernel(dst_hbm, ea_hbm, out_hbm, agg_sh, idx_a,
                       rows_a, zbuf):
        cid = lax.axis_index("c")
        sid = lax.axis_index("s")
        wid = cid * _NUM_SUBCORES + sid

        def load(j, idx_v, rows_v):
            pltpu.sync_copy(dst_hbm.at[pl.ds(j * _CHUNK, _CHUNK)], idx_v)
            pltpu.sync_copy(ea_hbm.at[j], rows_v)

        def scatter(idx_v, rows_v):
            pltpu.sync_copy(rows_v, agg_sh.at[idx_v], add=True)

        zero_row = jnp.zeros((d_edge,), jnp.float32)

        def zero_body(i, carry):
            zbuf[i] = zero_row
            return carry

        lax.fori_loop(0, rows_per_sub, zero_body, 0)
        row0 = sid * rows_per_sub
        pltpu.sync_copy(zbuf, agg_sh.at[pl.ds(row0, rows_per_sub)])
        plsc.subcore_barrier()

        start = wid * chunks_per + jnp.minimum(wid, chunks_rem)
        count = chunks_per + jnp.where(wid < chunks_rem, 1, 0)

        def chunk_body(j, carry):
            load(j, idx_a, rows_a)
            scatter(idx_a, rows_a)
            return carry

        lax.fori_loop(start, start + count, chunk_body, 0)

        plsc.subcore_barrier()

        pltpu.sync_copy(agg_sh.at[pl.ds(row0, rows_per_sub)],
                        out_hbm.at[cid, pl.ds(row0, rows_per_sub)])

    ea3 = edge_attr.reshape(n_chunks, _CHUNK, d_edge)
    return scatter_kernel(dst_idx, ea3)


def _mlp_body(x_ref, a0_ref, a1_ref, u_ref, w1x_ref, w1a_ref, w1u_ref,
              b1_ref, g1_ref, be1_ref, w2_ref, b2_ref, g2_ref, be2_ref,
              w3_ref, b3_ref, out_ref):
    f32 = jnp.float32
    agg = a0_ref[...] + a1_ref[...]
    h = (jnp.dot(x_ref[...], w1x_ref[...], preferred_element_type=f32)
         + jnp.dot(agg, w1a_ref[...], preferred_element_type=f32)
         + jnp.dot(u_ref[...], w1u_ref[...], preferred_element_type=f32)
         + b1_ref[...])
    h = jnp.maximum(h, 0.0)
    mu = jnp.mean(h, axis=-1, keepdims=True)
    var = jnp.mean((h - mu) ** 2, axis=-1, keepdims=True)
    h = (h - mu) / jnp.sqrt(var + 1e-5) * g1_ref[...] + be1_ref[...]
    h = jnp.dot(h, w2_ref[...], preferred_element_type=f32) + b2_ref[...]
    h = jnp.maximum(h, 0.0)
    mu = jnp.mean(h, axis=-1, keepdims=True)
    var = jnp.mean((h - mu) ** 2, axis=-1, keepdims=True)
    h = (h - mu) / jnp.sqrt(var + 1e-5) * g2_ref[...] + be2_ref[...]
    out_ref[...] = (jnp.dot(h, w3_ref[...], preferred_element_type=f32)
                    + b3_ref[...])


def kernel(x, edge_index, edge_attr, u, batch, W1, b1, g1, be1,
           W2, b2, g2, be2, W3, b3):
    n, d_feat = x.shape
    d_edge = edge_attr.shape[1]
    d_u = u.shape[1]
    hidden = W1.shape[1]
    out_size = W3.shape[1]

    dst_idx = edge_index[1].astype(jnp.int32)
    n_pad = -(-n // 128) * 128
    partials = _sc_scatter_partials(dst_idx, edge_attr, n_pad)
    a0, a1 = partials[0, :n], partials[1, :n]

    w1x = W1[:d_feat]
    w1a = W1[d_feat:d_feat + d_edge]
    w1u = W1[d_feat + d_edge:]

    br = 1000 if n % 1000 == 0 else n
    grid = (n // br,)
    row_spec = lambda width: pl.BlockSpec((br, width), lambda i: (i, 0))
    full = lambda a: pl.BlockSpec(a.shape, lambda i: (0,) * a.ndim)

    b1r, g1r, be1r = b1.reshape(1, -1), g1.reshape(1, -1), be1.reshape(1, -1)
    b2r, g2r, be2r = b2.reshape(1, -1), g2.reshape(1, -1), be2.reshape(1, -1)
    b3r = b3.reshape(1, -1)

    return pl.pallas_call(
        _mlp_body,
        grid=grid,
        in_specs=[
            row_spec(d_feat), row_spec(d_edge), row_spec(d_edge),
            full(u), full(w1x), full(w1a), full(w1u),
            full(b1r), full(g1r), full(be1r),
            full(W2), full(b2r), full(g2r), full(be2r),
            full(W3), full(b3r),
        ],
        out_specs=pl.BlockSpec((br, out_size), lambda i: (i, 0)),
        out_shape=jax.ShapeDtypeStruct((n, out_size), jnp.float32),
    )(x, a0, a1, u, w1x, w1a, w1u, b1r, g1r, be1r,
      W2, b2r, g2r, be2r, W3, b3r)


# speedup vs baseline: 4.8973x; 1.3158x over previous
import functools

import jax
import jax.numpy as jnp
from jax import lax
from jax.experimental import pallas as pl
from jax.experimental.pallas import tpu as pltpu
from jax.experimental.pallas import tpu_sc as plsc

_NUM_CORES = 2
_NUM_SUBCORES = 16
_NW = _NUM_CORES * _NUM_SUBCORES
_CHUNK = 128


def _sc_scatter_partials(dst_idx, edge_attr, n_pad):
    e, d_edge = edge_attr.shape
    n_chunks = e // _CHUNK
    rows_per_sub = n_pad // _NUM_SUBCORES
    chunks_per, chunks_rem = divmod(n_chunks, _NW)

    mesh = plsc.VectorSubcoreMesh(
        core_axis_name="c", subcore_axis_name="s",
        num_cores=_NUM_CORES, num_subcores=_NUM_SUBCORES)

    pairs = chunks_per // 2
    odd = chunks_per % 2

    @functools.partial(
        pl.kernel,
        out_type=jax.ShapeDtypeStruct((_NUM_CORES, n_pad, d_edge),
                                      jnp.float32),
        mesh=mesh,
        compiler_params=pltpu.CompilerParams(use_tc_tiling_on_sc=False),
        scratch_types=[
            pltpu.VMEM_SHARED((n_pad, d_edge), jnp.float32),
            pltpu.VMEM((_CHUNK,), jnp.int32),
            pltpu.VMEM((_CHUNK,), jnp.int32),
            pltpu.VMEM((_CHUNK, d_edge), jnp.float32),
            pltpu.VMEM((_CHUNK, d_edge), jnp.float32),
            pltpu.VMEM((rows_per_sub, d_edge), jnp.float32),
            pltpu.SemaphoreType.DMA,
            pltpu.SemaphoreType.DMA,
            pltpu.SemaphoreType.DMA,
        ],
    )
    def scatter_kernel(dst_hbm, ea_hbm, out_hbm, agg_sh, idx_a, idx_b,
                       rows_a, rows_b, zbuf, sem_la, sem_lb, sem_sc):
        cid = lax.axis_index("c")
        sid = lax.axis_index("s")
        wid = cid * _NUM_SUBCORES + sid

        def load(j, idx_v, rows_v, sem):
            pltpu.async_copy(dst_hbm.at[pl.ds(j * _CHUNK, _CHUNK)], idx_v,
                             sem)
            pltpu.async_copy(ea_hbm.at[j], rows_v, sem)

        def wait_load(idx_v, rows_v, sem):
            pltpu.make_async_copy(dst_hbm.at[pl.ds(0, _CHUNK)], idx_v,
                                  sem).wait()
            pltpu.make_async_copy(ea_hbm.at[0], rows_v, sem).wait()

        def scatter(idx_v, rows_v):
            pltpu.async_copy(rows_v, agg_sh.at[idx_v], sem_sc,
                             add=True).wait()

        zero_row = jnp.zeros((d_edge,), jnp.float32)

        def zero_body(i, carry):
            zbuf[i] = zero_row
            return carry

        lax.fori_loop(0, rows_per_sub, zero_body, 0)
        row0 = sid * rows_per_sub
        pltpu.sync_copy(zbuf, agg_sh.at[pl.ds(row0, rows_per_sub)])
        plsc.subcore_barrier()

        start = wid * chunks_per
        last_a = start + 2 * (pairs - 1)

        if pairs > 0:
            load(start, idx_a, rows_a, sem_la)
            load(start + 1, idx_b, rows_b, sem_lb)

        def pair_body(p, carry):
            j = start + 2 * p
            wait_load(idx_a, rows_a, sem_la)
            scatter(idx_a, rows_a)
            load(jnp.minimum(j + 2, last_a), idx_a, rows_a, sem_la)

            wait_load(idx_b, rows_b, sem_lb)
            scatter(idx_b, rows_b)
            load(jnp.minimum(j + 3, last_a + 1), idx_b, rows_b, sem_lb)

            return carry

        if pairs > 0:
            lax.fori_loop(0, pairs, pair_body, 0)
            wait_load(idx_a, rows_a, sem_la)
            wait_load(idx_b, rows_b, sem_lb)

        def tail_chunk(j):
            pltpu.sync_copy(dst_hbm.at[pl.ds(j * _CHUNK, _CHUNK)], idx_a)
            pltpu.sync_copy(ea_hbm.at[j], rows_a)
            pltpu.sync_copy(rows_a, agg_sh.at[idx_a], add=True)

        if odd:
            tail_chunk(start + chunks_per - 1)
        if chunks_rem:
            def rem_body(i, carry):
                tail_chunk(_NW * chunks_per + wid)
                return carry

            lax.fori_loop(0, jnp.where(wid < chunks_rem, 1, 0), rem_body, 0)

        plsc.subcore_barrier()

        pltpu.sync_copy(agg_sh.at[pl.ds(row0, rows_per_sub)],
                        out_hbm.at[cid, pl.ds(row0, rows_per_sub)])

    ea3 = edge_attr.reshape(n_chunks, _CHUNK, d_edge)
    return scatter_kernel(dst_idx, ea3)


def _mlp_body(x_ref, a0_ref, a1_ref, u_ref, w1x_ref, w1a_ref, w1u_ref,
              b1_ref, g1_ref, be1_ref, w2_ref, b2_ref, g2_ref, be2_ref,
              w3_ref, b3_ref, out_ref):
    f32 = jnp.float32
    agg = a0_ref[...] + a1_ref[...]
    h = (jnp.dot(x_ref[...], w1x_ref[...], preferred_element_type=f32)
         + jnp.dot(agg, w1a_ref[...], preferred_element_type=f32)
         + jnp.dot(u_ref[...], w1u_ref[...], preferred_element_type=f32)
         + b1_ref[...])
    h = jnp.maximum(h, 0.0)
    mu = jnp.mean(h, axis=-1, keepdims=True)
    var = jnp.mean((h - mu) ** 2, axis=-1, keepdims=True)
    h = (h - mu) / jnp.sqrt(var + 1e-5) * g1_ref[...] + be1_ref[...]
    h = jnp.dot(h, w2_ref[...], preferred_element_type=f32) + b2_ref[...]
    h = jnp.maximum(h, 0.0)
    mu = jnp.mean(h, axis=-1, keepdims=True)
    var = jnp.mean((h - mu) ** 2, axis=-1, keepdims=True)
    h = (h - mu) / jnp.sqrt(var + 1e-5) * g2_ref[...] + be2_ref[...]
    out_ref[...] = (jnp.dot(h, w3_ref[...], preferred_element_type=f32)
                    + b3_ref[...])


def kernel(x, edge_index, edge_attr, u, batch, W1, b1, g1, be1,
           W2, b2, g2, be2, W3, b3):
    n, d_feat = x.shape
    d_edge = edge_attr.shape[1]
    d_u = u.shape[1]
    hidden = W1.shape[1]
    out_size = W3.shape[1]

    dst_idx = edge_index[1].astype(jnp.int32)
    n_pad = -(-n // 128) * 128
    partials = _sc_scatter_partials(dst_idx, edge_attr, n_pad)
    a0, a1 = partials[0, :n], partials[1, :n]

    w1x = W1[:d_feat]
    w1a = W1[d_feat:d_feat + d_edge]
    w1u = W1[d_feat + d_edge:]

    br = 1000 if n % 1000 == 0 else n
    grid = (n // br,)
    row_spec = lambda width: pl.BlockSpec((br, width), lambda i: (i, 0))
    full = lambda a: pl.BlockSpec(a.shape, lambda i: (0,) * a.ndim)

    b1r, g1r, be1r = b1.reshape(1, -1), g1.reshape(1, -1), be1.reshape(1, -1)
    b2r, g2r, be2r = b2.reshape(1, -1), g2.reshape(1, -1), be2.reshape(1, -1)
    b3r = b3.reshape(1, -1)

    return pl.pallas_call(
        _mlp_body,
        grid=grid,
        in_specs=[
            row_spec(d_feat), row_spec(d_edge), row_spec(d_edge),
            full(u), full(w1x), full(w1a), full(w1u),
            full(b1r), full(g1r), full(be1r),
            full(W2), full(b2r), full(g2r), full(be2r),
            full(W3), full(b3r),
        ],
        out_specs=pl.BlockSpec((br, out_size), lambda i: (i, 0)),
        out_shape=jax.ShapeDtypeStruct((n, out_size), jnp.float32),
    )(x, a0, a1, u, w1x, w1a, w1u, b1r, g1r, be1r,
      W2, b2r, g2r, be2r, W3, b3r)


# speedup vs baseline: 5.0531x; 1.0318x over previous
import functools

import jax
import jax.numpy as jnp
from jax import lax
from jax.experimental import pallas as pl
from jax.experimental.pallas import tpu as pltpu
from jax.experimental.pallas import tpu_sc as plsc

_NUM_CORES = 2
_NUM_SUBCORES = 16
_NW = _NUM_CORES * _NUM_SUBCORES
_CHUNK = 128


def _sc_scatter_partials(dst_idx, edge_attr, n_pad):
    e, d_edge = edge_attr.shape
    n_chunks = e // _CHUNK
    rows_per_sub = n_pad // _NUM_SUBCORES
    chunks_per, chunks_rem = divmod(n_chunks, _NW)

    mesh = plsc.VectorSubcoreMesh(
        core_axis_name="c", subcore_axis_name="s",
        num_cores=_NUM_CORES, num_subcores=_NUM_SUBCORES)

    pairs = chunks_per // 2
    odd = chunks_per % 2

    @functools.partial(
        pl.kernel,
        out_type=jax.ShapeDtypeStruct((_NUM_CORES, n_pad, d_edge),
                                      jnp.float32),
        mesh=mesh,
        compiler_params=pltpu.CompilerParams(use_tc_tiling_on_sc=False),
        scratch_types=[
            pltpu.VMEM_SHARED((n_pad, d_edge), jnp.float32),
            pltpu.VMEM((_CHUNK,), jnp.int32),
            pltpu.VMEM((_CHUNK,), jnp.int32),
            pltpu.VMEM((_CHUNK, d_edge), jnp.float32),
            pltpu.VMEM((_CHUNK, d_edge), jnp.float32),
            pltpu.VMEM((rows_per_sub, d_edge), jnp.float32),
            pltpu.SemaphoreType.DMA,
            pltpu.SemaphoreType.DMA,
            pltpu.SemaphoreType.DMA,
        ],
    )
    def scatter_kernel(dst_hbm, ea_hbm, out_hbm, agg_sh, idx_a, idx_b,
                       rows_a, rows_b, zbuf, sem_la, sem_lb, sem_sc):
        cid = lax.axis_index("c")
        sid = lax.axis_index("s")
        wid = cid * _NUM_SUBCORES + sid

        def load(j, idx_v, rows_v, sem):
            pltpu.async_copy(dst_hbm.at[pl.ds(j * _CHUNK, _CHUNK)], idx_v,
                             sem)
            pltpu.async_copy(ea_hbm.at[j], rows_v, sem)

        def wait_load(idx_v, rows_v, sem):
            pltpu.make_async_copy(dst_hbm.at[pl.ds(0, _CHUNK)], idx_v,
                                  sem).wait()
            pltpu.make_async_copy(ea_hbm.at[0], rows_v, sem).wait()

        def scatter(idx_v, rows_v):
            pltpu.async_copy(rows_v, agg_sh.at[idx_v], sem_sc,
                             add=True).wait()

        zero_row = jnp.zeros((d_edge,), jnp.float32)

        def zero_body(i, carry):
            zbuf[i] = zero_row
            return carry

        lax.fori_loop(0, rows_per_sub, zero_body, 0)
        row0 = sid * rows_per_sub
        pltpu.sync_copy(zbuf, agg_sh.at[pl.ds(row0, rows_per_sub)])
        plsc.subcore_barrier()

        start = wid * chunks_per
        last_a = start + 2 * (pairs - 1)

        if pairs > 0:
            load(start, idx_a, rows_a, sem_la)
            load(start + 1, idx_b, rows_b, sem_lb)

        def pair_body(p, carry):
            j = start + 2 * p
            wait_load(idx_a, rows_a, sem_la)
            scatter(idx_a, rows_a)
            load(jnp.minimum(j + 2, last_a), idx_a, rows_a, sem_la)

            wait_load(idx_b, rows_b, sem_lb)
            scatter(idx_b, rows_b)
            load(jnp.minimum(j + 3, last_a + 1), idx_b, rows_b, sem_lb)

            return carry

        if pairs > 0:
            lax.fori_loop(0, pairs, pair_body, 0)
            wait_load(idx_a, rows_a, sem_la)
            wait_load(idx_b, rows_b, sem_lb)

        def tail_chunk(j):
            pltpu.sync_copy(dst_hbm.at[pl.ds(j * _CHUNK, _CHUNK)], idx_a)
            pltpu.sync_copy(ea_hbm.at[j], rows_a)
            pltpu.sync_copy(rows_a, agg_sh.at[idx_a], add=True)

        if odd:
            tail_chunk(start + chunks_per - 1)
        if chunks_rem:
            def rem_body(i, carry):
                tail_chunk(_NW * chunks_per + wid)
                return carry

            lax.fori_loop(0, jnp.where(wid < chunks_rem, 1, 0), rem_body, 0)

        plsc.subcore_barrier()

        pltpu.sync_copy(agg_sh.at[pl.ds(row0, rows_per_sub)],
                        out_hbm.at[cid, pl.ds(row0, rows_per_sub)])

    ea3 = edge_attr.reshape(n_chunks, _CHUNK, d_edge)
    return scatter_kernel(dst_idx, ea3)


def _mlp_body(x_ref, a0_ref, a1_ref, u_ref, w1x_ref, w1a_ref, w1u_ref,
              b1_ref, g1_ref, be1_ref, w2_ref, b2_ref, g2_ref, be2_ref,
              w3_ref, b3_ref, out_ref):
    f32 = jnp.float32
    agg = a0_ref[0] + a1_ref[0]
    h = (jnp.dot(x_ref[...], w1x_ref[...], preferred_element_type=f32)
         + jnp.dot(agg, w1a_ref[...], preferred_element_type=f32)
         + jnp.dot(u_ref[...], w1u_ref[...], preferred_element_type=f32)
         + b1_ref[...])
    h = jnp.maximum(h, 0.0)
    mu = jnp.mean(h, axis=-1, keepdims=True)
    var = jnp.mean((h - mu) ** 2, axis=-1, keepdims=True)
    h = (h - mu) / jnp.sqrt(var + 1e-5) * g1_ref[...] + be1_ref[...]
    h = jnp.dot(h, w2_ref[...], preferred_element_type=f32) + b2_ref[...]
    h = jnp.maximum(h, 0.0)
    mu = jnp.mean(h, axis=-1, keepdims=True)
    var = jnp.mean((h - mu) ** 2, axis=-1, keepdims=True)
    h = (h - mu) / jnp.sqrt(var + 1e-5) * g2_ref[...] + be2_ref[...]
    out_ref[...] = (jnp.dot(h, w3_ref[...], preferred_element_type=f32)
                    + b3_ref[...])


def kernel(x, edge_index, edge_attr, u, batch, W1, b1, g1, be1,
           W2, b2, g2, be2, W3, b3):
    n, d_feat = x.shape
    d_edge = edge_attr.shape[1]
    d_u = u.shape[1]
    hidden = W1.shape[1]
    out_size = W3.shape[1]

    dst_idx = edge_index[1].astype(jnp.int32)
    n_pad = -(-n // 128) * 128
    partials = _sc_scatter_partials(dst_idx, edge_attr, n_pad)

    w1x = W1[:d_feat]
    w1a = W1[d_feat:d_feat + d_edge]
    w1u = W1[d_feat + d_edge:]

    br = 1000 if n % 1000 == 0 else n
    grid = (n // br,)
    row_spec = lambda width: pl.BlockSpec((br, width), lambda i: (i, 0))
    full = lambda a: pl.BlockSpec(a.shape, lambda i: (0,) * a.ndim)

    b1r, g1r, be1r = b1.reshape(1, -1), g1.reshape(1, -1), be1.reshape(1, -1)
    b2r, g2r, be2r = b2.reshape(1, -1), g2.reshape(1, -1), be2.reshape(1, -1)
    b3r = b3.reshape(1, -1)

    part_spec = lambda c: pl.BlockSpec((1, br, d_edge),
                                       lambda i, c=c: (c, i, 0))

    return pl.pallas_call(
        _mlp_body,
        grid=grid,
        in_specs=[
            row_spec(d_feat), part_spec(0), part_spec(1),
            full(u), full(w1x), full(w1a), full(w1u),
            full(b1r), full(g1r), full(be1r),
            full(W2), full(b2r), full(g2r), full(be2r),
            full(W3), full(b3r),
        ],
        out_specs=pl.BlockSpec((br, out_size), lambda i: (i, 0)),
        out_shape=jax.ShapeDtypeStruct((n, out_size), jnp.float32),
    )(x, partials, partials, u, w1x, w1a, w1u, b1r, g1r, be1r,
      W2, b2r, g2r, be2r, W3, b3r)


# speedup vs baseline: 5.0650x; 1.0023x over previous
import functools

import jax
import jax.numpy as jnp
from jax import lax
from jax.experimental import pallas as pl
from jax.experimental.pallas import tpu as pltpu
from jax.experimental.pallas import tpu_sc as plsc

_NUM_CORES = 2
_NUM_SUBCORES = 16
_NW = _NUM_CORES * _NUM_SUBCORES
_CHUNK = 128


def _sc_scatter_partials(dst_idx, edge_attr, n_pad):
    e, d_edge = edge_attr.shape
    n_chunks = e // _CHUNK
    rows_per_sub = n_pad // _NUM_SUBCORES
    chunks_per, chunks_rem = divmod(n_chunks, _NW)

    mesh = plsc.VectorSubcoreMesh(
        core_axis_name="c", subcore_axis_name="s",
        num_cores=_NUM_CORES, num_subcores=_NUM_SUBCORES)

    pairs = chunks_per // 2
    odd = chunks_per % 2

    @functools.partial(
        pl.kernel,
        out_type=jax.ShapeDtypeStruct((_NUM_CORES, n_pad, d_edge),
                                      jnp.float32),
        mesh=mesh,
        compiler_params=pltpu.CompilerParams(use_tc_tiling_on_sc=False),
        scratch_types=[
            pltpu.VMEM_SHARED((n_pad, d_edge), jnp.float32),
            pltpu.VMEM((_CHUNK,), jnp.int32),
            pltpu.VMEM((_CHUNK,), jnp.int32),
            pltpu.VMEM((_CHUNK, d_edge), jnp.float32),
            pltpu.VMEM((_CHUNK, d_edge), jnp.float32),
            pltpu.VMEM((rows_per_sub, d_edge), jnp.float32),
            pltpu.SemaphoreType.DMA,
            pltpu.SemaphoreType.DMA,
            pltpu.SemaphoreType.DMA,
        ],
    )
    def scatter_kernel(dst_hbm, ea_hbm, out_hbm, agg_sh, idx_a, idx_b,
                       rows_a, rows_b, zbuf, sem_la, sem_lb, sem_sc):
        cid = lax.axis_index("c")
        sid = lax.axis_index("s")
        wid = cid * _NUM_SUBCORES + sid

        def load(j, idx_v, rows_v, sem):
            pltpu.async_copy(dst_hbm.at[pl.ds(j * _CHUNK, _CHUNK)], idx_v,
                             sem)
            pltpu.async_copy(ea_hbm.at[pl.ds(j * _CHUNK, _CHUNK)], rows_v,
                             sem)

        def wait_load(idx_v, rows_v, sem):
            pltpu.make_async_copy(dst_hbm.at[pl.ds(0, _CHUNK)], idx_v,
                                  sem).wait()
            pltpu.make_async_copy(ea_hbm.at[pl.ds(0, _CHUNK)], rows_v,
                                  sem).wait()

        def scatter(idx_v, rows_v):
            pltpu.async_copy(rows_v, agg_sh.at[idx_v], sem_sc,
                             add=True).wait()

        zero_row = jnp.zeros((d_edge,), jnp.float32)

        def zero_body(i, carry):
            zbuf[i] = zero_row
            return carry

        lax.fori_loop(0, rows_per_sub, zero_body, 0)
        row0 = sid * rows_per_sub
        pltpu.sync_copy(zbuf, agg_sh.at[pl.ds(row0, rows_per_sub)])
        plsc.subcore_barrier()

        start = wid * chunks_per
        last_a = start + 2 * (pairs - 1)

        if pairs > 0:
            load(start, idx_a, rows_a, sem_la)
            load(start + 1, idx_b, rows_b, sem_lb)

        def pair_body(p, carry):
            j = start + 2 * p
            wait_load(idx_a, rows_a, sem_la)
            scatter(idx_a, rows_a)
            load(jnp.minimum(j + 2, last_a), idx_a, rows_a, sem_la)

            wait_load(idx_b, rows_b, sem_lb)
            scatter(idx_b, rows_b)
            load(jnp.minimum(j + 3, last_a + 1), idx_b, rows_b, sem_lb)

            return carry

        if pairs > 0:
            lax.fori_loop(0, pairs, pair_body, 0)
            wait_load(idx_a, rows_a, sem_la)
            wait_load(idx_b, rows_b, sem_lb)

        def tail_chunk(j):
            pltpu.sync_copy(dst_hbm.at[pl.ds(j * _CHUNK, _CHUNK)], idx_a)
            pltpu.sync_copy(ea_hbm.at[pl.ds(j * _CHUNK, _CHUNK)], rows_a)
            pltpu.sync_copy(rows_a, agg_sh.at[idx_a], add=True)

        if odd:
            tail_chunk(start + chunks_per - 1)
        if chunks_rem:
            def rem_body(i, carry):
                tail_chunk(_NW * chunks_per + wid)
                return carry

            lax.fori_loop(0, jnp.where(wid < chunks_rem, 1, 0), rem_body, 0)

        plsc.subcore_barrier()

        pltpu.sync_copy(agg_sh.at[pl.ds(row0, rows_per_sub)],
                        out_hbm.at[cid, pl.ds(row0, rows_per_sub)])

    return scatter_kernel(dst_idx, edge_attr)


def _mlp_body(x_ref, a0_ref, a1_ref, u_ref, w1x_ref, w1a_ref, w1u_ref,
              b1_ref, g1_ref, be1_ref, w2_ref, b2_ref, g2_ref, be2_ref,
              w3_ref, b3_ref, out_ref):
    f32 = jnp.float32
    agg = a0_ref[0] + a1_ref[0]
    h = (jnp.dot(x_ref[...], w1x_ref[...], preferred_element_type=f32)
         + jnp.dot(agg, w1a_ref[...], preferred_element_type=f32)
         + jnp.dot(u_ref[...], w1u_ref[...], preferred_element_type=f32)
         + b1_ref[...])
    h = jnp.maximum(h, 0.0)
    mu = jnp.mean(h, axis=-1, keepdims=True)
    var = jnp.mean((h - mu) ** 2, axis=-1, keepdims=True)
    h = (h - mu) / jnp.sqrt(var + 1e-5) * g1_ref[...] + be1_ref[...]
    h = jnp.dot(h, w2_ref[...], preferred_element_type=f32) + b2_ref[...]
    h = jnp.maximum(h, 0.0)
    mu = jnp.mean(h, axis=-1, keepdims=True)
    var = jnp.mean((h - mu) ** 2, axis=-1, keepdims=True)
    h = (h - mu) / jnp.sqrt(var + 1e-5) * g2_ref[...] + be2_ref[...]
    out_ref[...] = (jnp.dot(h, w3_ref[...], preferred_element_type=f32)
                    + b3_ref[...])


def kernel(x, edge_index, edge_attr, u, batch, W1, b1, g1, be1,
           W2, b2, g2, be2, W3, b3):
    n, d_feat = x.shape
    d_edge = edge_attr.shape[1]
    d_u = u.shape[1]
    hidden = W1.shape[1]
    out_size = W3.shape[1]

    dst_idx = edge_index[1].astype(jnp.int32)
    n_pad = -(-n // 128) * 128
    partials = _sc_scatter_partials(dst_idx, edge_attr, n_pad)

    w1x = W1[:d_feat]
    w1a = W1[d_feat:d_feat + d_edge]
    w1u = W1[d_feat + d_edge:]

    br = 1000 if n % 1000 == 0 else n
    grid = (n // br,)
    row_spec = lambda width: pl.BlockSpec((br, width), lambda i: (i, 0))
    full = lambda a: pl.BlockSpec(a.shape, lambda i: (0,) * a.ndim)

    b1r, g1r, be1r = b1.reshape(1, -1), g1.reshape(1, -1), be1.reshape(1, -1)
    b2r, g2r, be2r = b2.reshape(1, -1), g2.reshape(1, -1), be2.reshape(1, -1)
    b3r = b3.reshape(1, -1)

    part_spec = lambda c: pl.BlockSpec((1, br, d_edge),
                                       lambda i, c=c: (c, i, 0))

    return pl.pallas_call(
        _mlp_body,
        grid=grid,
        in_specs=[
            row_spec(d_feat), part_spec(0), part_spec(1),
            full(u), full(w1x), full(w1a), full(w1u),
            full(b1r), full(g1r), full(be1r),
            full(W2), full(b2r), full(g2r), full(be2r),
            full(W3), full(b3r),
        ],
        out_specs=pl.BlockSpec((br, out_size), lambda i: (i, 0)),
        out_shape=jax.ShapeDtypeStruct((n, out_size), jnp.float32),
    )(x, partials, partials, u, w1x, w1a, w1u, b1r, g1r, be1r,
      W2, b2r, g2r, be2r, W3, b3r)
